# baseline (speedup 1.0000x reference)
import functools
import math

import jax
import jax.numpy as jnp
from jax import lax
from jax.experimental import pallas as pl
from jax.experimental.pallas import tpu as pltpu
from jax.experimental.pallas import tpu_sc as plsc

B, N, DEG, D, H = 2, 10000, 16, 128, 4
HD = D // H
NW = 32
CH = 128
TW = 2 * D

NSLAB = 5120
NPADH = DEG * NSLAB
ROWS_PER_W = NPADH // NW
NCH = ROWS_PER_W // CH

NB = 128
GB = NB * DEG
NBLK = NSLAB // NB

NBP = 2000
NPRE = N // NBP


def _pack_bf16(y):
    u = lax.bitcast_convert_type(y, jnp.uint32)
    r = (u + jnp.uint32(0x8000)) >> jnp.uint32(16)
    packed = r[:, :D // 2] | (r[:, D // 2:] << jnp.uint32(16))
    return lax.bitcast_convert_type(packed, jnp.int32)


def _unpack_bf16(gi):
    gu = lax.bitcast_convert_type(gi, jnp.uint32)
    lo = lax.bitcast_convert_type(gu << jnp.uint32(16), jnp.float32)
    hi = lax.bitcast_convert_type(gu & jnp.uint32(0xFFFF0000), jnp.float32)
    return jnp.concatenate([lo, hi], axis=1)


def _pre_body(x_ref, wkT_ref, bk_ref, wvT_ref, bv_ref, out_ref):
    wkT = wkT_ref[...]
    wvT = wvT_ref[...]
    hw = D // 2
    for b in range(B):
        xb = x_ref[b]
        out_ref[:, b * hw:(b + 1) * hw] = _pack_bf16(
            jnp.dot(xb, wkT, preferred_element_type=jnp.float32) + bk_ref[...])
        out_ref[:, D + b * hw:D + (b + 1) * hw] = _pack_bf16(
            jnp.dot(xb, wvT, preferred_element_type=jnp.float32) + bv_ref[...])


def _build_table(x, WkT, bk, WvT, bv):
    return pl.pallas_call(
        _pre_body,
        grid=(NPRE,),
        in_specs=[
            pl.BlockSpec((B, NBP, D), lambda i: (0, i, 0)),
            pl.BlockSpec((D, D), lambda i: (0, 0)),
            pl.BlockSpec((1, D), lambda i: (0, 0)),
            pl.BlockSpec((D, D), lambda i: (0, 0)),
            pl.BlockSpec((1, D), lambda i: (0, 0)),
        ],
        out_specs=pl.BlockSpec((NBP, TW), lambda i: (i, 0)),
        out_shape=jax.ShapeDtypeStruct((N, TW), jnp.int32),
    )(x, WkT, bk, WvT, bv)


def _gather_body(table_hbm, idx_hbm, out_hbm, idx_v, rows_v, sem0, sem1):
    c = lax.axis_index("c")
    s = lax.axis_index("s")
    wid = s * 2 + c
    base = wid * ROWS_PER_W
    pltpu.sync_copy(idx_hbm.at[pl.ds(base, ROWS_PER_W)], idx_v)
    sems = (sem0, sem1)
    bufs = (rows_v.at[0], rows_v.at[1])

    def start_g(j, b):
        pltpu.async_copy(
            table_hbm.at[idx_v.at[pl.ds(j * CH, CH)]], bufs[b], sems[b])

    def finish(j, b):
        pltpu.make_async_copy(
            table_hbm.at[idx_v.at[pl.ds(0, CH)]], bufs[b], sems[b]).wait()
        pltpu.sync_copy(bufs[b], out_hbm.at[pl.ds(base + j * CH, CH)])

    start_g(0, 0)

    def body(p, carry):
        j0 = p * 2
        start_g(j0 + 1, 1)
        finish(j0, 0)

        @pl.when(p < NCH // 2 - 1)
        def _():
            start_g(j0 + 2, 0)

        finish(j0 + 1, 1)
        return carry

    lax.fori_loop(0, NCH // 2, body, 0)


def _gather_rows(table, idx_pad):
    mesh = plsc.VectorSubcoreMesh(core_axis_name="c", subcore_axis_name="s")
    k = pl.kernel(
        _gather_body,
        out_type=jax.ShapeDtypeStruct((NPADH, TW), jnp.int32),
        mesh=mesh,
        scratch_types=[
            pltpu.VMEM((ROWS_PER_W,), jnp.int32),
            pltpu.VMEM((2, CH, TW), jnp.int32),
            pltpu.SemaphoreType.DMA,
            pltpu.SemaphoreType.DMA,
        ],
    )
    return k(table, idx_pad)


def _post_body(x_ref, g_ref, dir_ref, mask_ref,
               wqT_ref, bq_ref, woT_ref, bo_ref,
               ln1g_ref, ln1b_ref, ln2g_ref, ln2b_ref,
               wf1T_ref, bf1_ref, wf2T_ref, bf2_ref,
               weC_ref, beC_ref, p_ref, out_ref):
    inv = 1.0 / math.sqrt(HD)
    hw = D // 2
    P = p_ref[...]
    P_lo = P[:hw, :]
    P_hi = P[hw:, :]
    woT = woT_ref[...]

    dirS = dir_ref[...].reshape(GB, 1)
    maskS = mask_ref[...].reshape(GB, 1)
    em = jnp.where(maskS > 0.5,
                   dirS * weC_ref[...] + beC_ref[...], -1e9)
    wqT = wqT_ref[...]
    gb = g_ref[...].reshape(GB, TW)

    def _halves(u):
        uu = lax.bitcast_convert_type(u, jnp.uint32)
        lo = lax.bitcast_convert_type(uu << jnp.uint32(16), jnp.float32)
        hi = lax.bitcast_convert_type(
            uu & jnp.uint32(0xFFFF0000), jnp.float32)
        return lo, hi

    def _slabsum(a):
        parts = [a[d * NB:(d + 1) * NB, :] for d in range(DEG)]
        while len(parts) > 1:
            parts = [parts[i] + parts[i + 1] for i in range(0, len(parts), 2)]
        return parts[0]

    for b in range(B):
        xb = x_ref[b]
        q = (jnp.dot(xb, wqT, preferred_element_type=jnp.float32)
             + bq_ref[...]) * inv
        q_lo = q[:, :hw]
        q_hi = q[:, hw:]
        kl, kh = _halves(gb[:, b * hw:(b + 1) * hw])
        prod_lo = (kl.reshape(DEG, NB, hw) * q_lo[None]).reshape(GB, hw)
        prod_hi = (kh.reshape(DEG, NB, hw) * q_hi[None]).reshape(GB, hw)
        lg = (jnp.dot(prod_lo, P_lo, preferred_element_type=jnp.float32)
              + jnp.dot(prod_hi, P_hi, preferred_element_type=jnp.float32)
              + em)
        e = jnp.exp(lg)
        vl, vh = _halves(gb[:, D + b * hw:D + (b + 1) * hw])
        wl = e[:, :hw] * vl
        wh = e[:, hw:] * vh
        den = _slabsum(e) + 1e-20
        agg_lo = _slabsum(wl) / den[:, :hw]
        agg_hi = _slabsum(wh) / den[:, hw:]

        h1 = (xb
              + jnp.dot(agg_lo, woT[:hw, :],
                        preferred_element_type=jnp.float32)
              + jnp.dot(agg_hi, woT[hw:, :],
                        preferred_element_type=jnp.float32) + bo_ref[...])
        m = jnp.mean(h1, axis=-1, keepdims=True)
        v = jnp.mean((h1 - m) ** 2, axis=-1, keepdims=True)
        h = (h1 - m) / jnp.sqrt(v + 1e-5) * ln1g_ref[...] + ln1b_ref[...]

        f = jnp.maximum(
            jnp.dot(h, wf1T_ref[...], preferred_element_type=jnp.float32)
            + bf1_ref[...], 0.0)
        f = jnp.dot(f, wf2T_ref[...],
                    preferred_element_type=jnp.float32) + bf2_ref[...]
        h2 = h + f
        m2 = jnp.mean(h2, axis=-1, keepdims=True)
        v2 = jnp.mean((h2 - m2) ** 2, axis=-1, keepdims=True)
        out_ref[b] = ((h2 - m2) / jnp.sqrt(v2 + 1e-5) * ln2g_ref[...]
                      + ln2b_ref[...])


def _post(x, g, dirE, maskE, WqT, bq, WoT, bo, ln1g, ln1b, ln2g, ln2b,
          Wf1T, bf1, Wf2T, bf2, weC, beC, P):
    full = lambda i: (0, 0)
    return pl.pallas_call(
        _post_body,
        grid=(NBLK,),
        in_specs=[
            pl.BlockSpec((B, NB, D), lambda i: (0, i, 0)),
            pl.BlockSpec((DEG, NB, TW), lambda i: (0, i, 0)),
            pl.BlockSpec((DEG, NB, 1), lambda i: (0, i, 0)),
            pl.BlockSpec((DEG, NB, 1), lambda i: (0, i, 0)),
            pl.BlockSpec((D, D), full),
            pl.BlockSpec((1, D), full),
            pl.BlockSpec((D, D), full),
            pl.BlockSpec((1, D), full),
            pl.BlockSpec((1, D), full),
            pl.BlockSpec((1, D), full),
            pl.BlockSpec((1, D), full),
            pl.BlockSpec((1, D), full),
            pl.BlockSpec((D, 2 * D), full),
            pl.BlockSpec((1, 2 * D), full),
            pl.BlockSpec((2 * D, D), full),
            pl.BlockSpec((1, D), full),
            pl.BlockSpec((1, D), full),
            pl.BlockSpec((1, D), full),
            pl.BlockSpec((D, D), lambda i: (0, 0)),
        ],
        out_specs=pl.BlockSpec((B, NB, D), lambda i: (0, i, 0)),
        out_shape=jax.ShapeDtypeStruct((B, NSLAB, D), jnp.float32),
    )(x, g, dirE, maskE, WqT, bq, WoT, bo, ln1g, ln1b, ln2g, ln2b,
      Wf1T, bf1, Wf2T, bf2, weC, beC, P)


def kernel(x, incoming_idx, incoming_dir, incoming_mask,
           Wq, bq, Wk, bk, Wv, bv, We, be, Wo, bo,
           ln1_g, ln1_b, ln2_g, ln2_b, Wf1, bf1, Wf2, bf2):
    table = _build_table(x, Wk.T, bk[None, :], Wv.T, bv[None, :])

    npad2 = 2 * NSLAB - N
    idxT = jnp.pad(incoming_idx.T, ((0, 0), (0, npad2)))
    dirT = jnp.pad(incoming_dir[:, :, 0].T, ((0, 0), (0, npad2)))
    maskT = jnp.pad(incoming_mask.T.astype(jnp.float32),
                    ((0, 0), (0, npad2)))
    xp = jnp.pad(x, ((0, 0), (0, npad2), (0, 0)))

    cols = jnp.arange(D, dtype=jnp.int32)
    P = (cols[:, None] // HD == cols[None, :] // HD).astype(jnp.float32)
    weC = jnp.repeat(We[:, 0], HD)[None, :]
    beC = jnp.repeat(be, HD)[None, :]

    outs = []
    for h0 in (0, NSLAB):
        sl = slice(h0, h0 + NSLAB)
        g = _gather_rows(table, idxT[:, sl].reshape(-1))
        g3 = g.reshape(DEG, NSLAB, TW)
        dir3 = dirT[:, sl].reshape(DEG, NSLAB, 1)
        mask3 = maskT[:, sl].reshape(DEG, NSLAB, 1)
        outs.append(_post(
            xp[:, sl], g3, dir3, mask3, Wq.T, bq[None, :], Wo.T,
            bo[None, :], ln1_g[None, :], ln1_b[None, :], ln2_g[None, :],
            ln2_b[None, :], Wf1.T, bf1[None, :], Wf2.T, bf2[None, :],
            weC, beC, P))

    return jnp.concatenate(outs, axis=1)[:, :N]


# speedup vs baseline: 1.0021x; 1.0021x over previous
import functools
import math

import jax
import jax.numpy as jnp
from jax import lax
from jax.experimental import pallas as pl
from jax.experimental.pallas import tpu as pltpu
from jax.experimental.pallas import tpu_sc as plsc

B, N, DEG, D, H = 2, 10000, 16, 128, 4
HD = D // H
NW = 32
CH = 64
TW = 2 * D

NSLAB = 5120
NPADH = DEG * NSLAB
ROWS_PER_W = NPADH // NW
NCH = ROWS_PER_W // CH

NB = 128
GB = NB * DEG
NBLK = NSLAB // NB

NBP = 2000
NPRE = N // NBP


def _pack_bf16(y):
    u = lax.bitcast_convert_type(y, jnp.uint32)
    r = (u + jnp.uint32(0x8000)) >> jnp.uint32(16)
    packed = r[:, :D // 2] | (r[:, D // 2:] << jnp.uint32(16))
    return lax.bitcast_convert_type(packed, jnp.int32)


def _unpack_bf16(gi):
    gu = lax.bitcast_convert_type(gi, jnp.uint32)
    lo = lax.bitcast_convert_type(gu << jnp.uint32(16), jnp.float32)
    hi = lax.bitcast_convert_type(gu & jnp.uint32(0xFFFF0000), jnp.float32)
    return jnp.concatenate([lo, hi], axis=1)


def _pre_body(x_ref, wkT_ref, bk_ref, wvT_ref, bv_ref, out_ref):
    wkT = wkT_ref[...]
    wvT = wvT_ref[...]
    hw = D // 2
    for b in range(B):
        xb = x_ref[b]
        out_ref[:, b * hw:(b + 1) * hw] = _pack_bf16(
            jnp.dot(xb, wkT, preferred_element_type=jnp.float32) + bk_ref[...])
        out_ref[:, D + b * hw:D + (b + 1) * hw] = _pack_bf16(
            jnp.dot(xb, wvT, preferred_element_type=jnp.float32) + bv_ref[...])


def _build_table(x, WkT, bk, WvT, bv):
    return pl.pallas_call(
        _pre_body,
        grid=(NPRE,),
        in_specs=[
            pl.BlockSpec((B, NBP, D), lambda i: (0, i, 0)),
            pl.BlockSpec((D, D), lambda i: (0, 0)),
            pl.BlockSpec((1, D), lambda i: (0, 0)),
            pl.BlockSpec((D, D), lambda i: (0, 0)),
            pl.BlockSpec((1, D), lambda i: (0, 0)),
        ],
        out_specs=pl.BlockSpec((NBP, TW), lambda i: (i, 0)),
        out_shape=jax.ShapeDtypeStruct((N, TW), jnp.int32),
    )(x, WkT, bk, WvT, bv)


NBUF = 6


def _gather_body(table_hbm, idx_hbm, out_hbm, idx_v, rows_v, *sems):
    gsems = sems[:NBUF]
    wsems = sems[NBUF:]
    c = lax.axis_index("c")
    s = lax.axis_index("s")
    wid = s * 2 + c
    base = wid * ROWS_PER_W
    pltpu.sync_copy(idx_hbm.at[pl.ds(base, ROWS_PER_W)], idx_v)
    bufs = [rows_v.at[i] for i in range(NBUF)]

    def start_g(j, b):
        pltpu.async_copy(
            table_hbm.at[idx_v.at[pl.ds(j * CH, CH)]], bufs[b], gsems[b])

    def wait_g(b):
        pltpu.make_async_copy(
            table_hbm.at[idx_v.at[pl.ds(0, CH)]], bufs[b], gsems[b]).wait()

    def start_w(j, b):
        pltpu.async_copy(
            bufs[b], out_hbm.at[pl.ds(base + j * CH, CH)], wsems[b])

    def wait_w(b):
        pltpu.make_async_copy(
            bufs[b], out_hbm.at[pl.ds(base, CH)], wsems[b]).wait()

    for b in range(NBUF - 1):
        start_g(b, b)
    for j in range(NCH):
        b = j % NBUF
        wait_g(b)
        start_w(j, b)
        jn = j + NBUF - 1
        if jn < NCH:
            bn = jn % NBUF
            if jn >= NBUF:
                wait_w(bn)
            start_g(jn, bn)
    for j in range(max(0, NCH - NBUF), NCH):
        wait_w(j % NBUF)


def _gather_rows(table, idx_pad):
    mesh = plsc.VectorSubcoreMesh(core_axis_name="c", subcore_axis_name="s")
    k = pl.kernel(
        _gather_body,
        out_type=jax.ShapeDtypeStruct((NPADH, TW), jnp.int32),
        mesh=mesh,
        scratch_types=(
            [pltpu.VMEM((ROWS_PER_W,), jnp.int32),
             pltpu.VMEM((NBUF, CH, TW), jnp.int32)]
            + [pltpu.SemaphoreType.DMA] * (2 * NBUF)
        ),
    )
    return k(table, idx_pad)


def _post_body(x_ref, g_ref, dir_ref, mask_ref,
               wqT_ref, bq_ref, woT_ref, bo_ref,
               ln1g_ref, ln1b_ref, ln2g_ref, ln2b_ref,
               wf1T_ref, bf1_ref, wf2T_ref, bf2_ref,
               weC_ref, beC_ref, p_ref, out_ref):
    inv = 1.0 / math.sqrt(HD)
    hw = D // 2
    P = p_ref[...]
    P_lo = P[:hw, :]
    P_hi = P[hw:, :]
    woT = woT_ref[...]

    dirS = dir_ref[...].reshape(GB, 1)
    maskS = mask_ref[...].reshape(GB, 1)
    em = jnp.where(maskS > 0.5,
                   dirS * weC_ref[...] + beC_ref[...], -1e9)
    wqT = wqT_ref[...]
    gb = g_ref[...].reshape(GB, TW)

    def _halves(u):
        uu = lax.bitcast_convert_type(u, jnp.uint32)
        lo = lax.bitcast_convert_type(uu << jnp.uint32(16), jnp.float32)
        hi = lax.bitcast_convert_type(
            uu & jnp.uint32(0xFFFF0000), jnp.float32)
        return lo, hi

    def _slabsum(a):
        parts = [a[d * NB:(d + 1) * NB, :] for d in range(DEG)]
        while len(parts) > 1:
            parts = [parts[i] + parts[i + 1] for i in range(0, len(parts), 2)]
        return parts[0]

    for b in range(B):
        xb = x_ref[b]
        q = (jnp.dot(xb, wqT, preferred_element_type=jnp.float32)
             + bq_ref[...]) * inv
        q_lo = q[:, :hw]
        q_hi = q[:, hw:]
        kl, kh = _halves(gb[:, b * hw:(b + 1) * hw])
        prod_lo = (kl.reshape(DEG, NB, hw) * q_lo[None]).reshape(GB, hw)
        prod_hi = (kh.reshape(DEG, NB, hw) * q_hi[None]).reshape(GB, hw)
        lg = (jnp.dot(prod_lo, P_lo, preferred_element_type=jnp.float32)
              + jnp.dot(prod_hi, P_hi, preferred_element_type=jnp.float32)
              + em)
        e = jnp.exp(lg)
        vl, vh = _halves(gb[:, D + b * hw:D + (b + 1) * hw])
        wl = e[:, :hw] * vl
        wh = e[:, hw:] * vh
        den = _slabsum(e) + 1e-20
        agg_lo = _slabsum(wl) / den[:, :hw]
        agg_hi = _slabsum(wh) / den[:, hw:]

        h1 = (xb
              + jnp.dot(agg_lo, woT[:hw, :],
                        preferred_element_type=jnp.float32)
              + jnp.dot(agg_hi, woT[hw:, :],
                        preferred_element_type=jnp.float32) + bo_ref[...])
        m = jnp.mean(h1, axis=-1, keepdims=True)
        v = jnp.mean((h1 - m) ** 2, axis=-1, keepdims=True)
        h = (h1 - m) / jnp.sqrt(v + 1e-5) * ln1g_ref[...] + ln1b_ref[...]

        f = jnp.maximum(
            jnp.dot(h, wf1T_ref[...], preferred_element_type=jnp.float32)
            + bf1_ref[...], 0.0)
        f = jnp.dot(f, wf2T_ref[...],
                    preferred_element_type=jnp.float32) + bf2_ref[...]
        h2 = h + f
        m2 = jnp.mean(h2, axis=-1, keepdims=True)
        v2 = jnp.mean((h2 - m2) ** 2, axis=-1, keepdims=True)
        out_ref[b] = ((h2 - m2) / jnp.sqrt(v2 + 1e-5) * ln2g_ref[...]
                      + ln2b_ref[...])


def _post(x, g, dirE, maskE, WqT, bq, WoT, bo, ln1g, ln1b, ln2g, ln2b,
          Wf1T, bf1, Wf2T, bf2, weC, beC, P):
    full = lambda i: (0, 0)
    return pl.pallas_call(
        _post_body,
        grid=(NBLK,),
        in_specs=[
            pl.BlockSpec((B, NB, D), lambda i: (0, i, 0)),
            pl.BlockSpec((DEG, NB, TW), lambda i: (0, i, 0)),
            pl.BlockSpec((DEG, NB, 1), lambda i: (0, i, 0)),
            pl.BlockSpec((DEG, NB, 1), lambda i: (0, i, 0)),
            pl.BlockSpec((D, D), full),
            pl.BlockSpec((1, D), full),
            pl.BlockSpec((D, D), full),
            pl.BlockSpec((1, D), full),
            pl.BlockSpec((1, D), full),
            pl.BlockSpec((1, D), full),
            pl.BlockSpec((1, D), full),
            pl.BlockSpec((1, D), full),
            pl.BlockSpec((D, 2 * D), full),
            pl.BlockSpec((1, 2 * D), full),
            pl.BlockSpec((2 * D, D), full),
            pl.BlockSpec((1, D), full),
            pl.BlockSpec((1, D), full),
            pl.BlockSpec((1, D), full),
            pl.BlockSpec((D, D), lambda i: (0, 0)),
        ],
        out_specs=pl.BlockSpec((B, NB, D), lambda i: (0, i, 0)),
        out_shape=jax.ShapeDtypeStruct((B, NSLAB, D), jnp.float32),
    )(x, g, dirE, maskE, WqT, bq, WoT, bo, ln1g, ln1b, ln2g, ln2b,
      Wf1T, bf1, Wf2T, bf2, weC, beC, P)


def kernel(x, incoming_idx, incoming_dir, incoming_mask,
           Wq, bq, Wk, bk, Wv, bv, We, be, Wo, bo,
           ln1_g, ln1_b, ln2_g, ln2_b, Wf1, bf1, Wf2, bf2):
    table = _build_table(x, Wk.T, bk[None, :], Wv.T, bv[None, :])

    npad2 = 2 * NSLAB - N
    idxT = jnp.pad(incoming_idx.T, ((0, 0), (0, npad2)))
    dirT = jnp.pad(incoming_dir[:, :, 0].T, ((0, 0), (0, npad2)))
    maskT = jnp.pad(incoming_mask.T.astype(jnp.float32),
                    ((0, 0), (0, npad2)))
    xp = jnp.pad(x, ((0, 0), (0, npad2), (0, 0)))

    cols = jnp.arange(D, dtype=jnp.int32)
    P = (cols[:, None] // HD == cols[None, :] // HD).astype(jnp.float32)
    weC = jnp.repeat(We[:, 0], HD)[None, :]
    beC = jnp.repeat(be, HD)[None, :]

    outs = []
    for h0 in (0, NSLAB):
        sl = slice(h0, h0 + NSLAB)
        g = _gather_rows(table, idxT[:, sl].reshape(-1))
        g3 = g.reshape(DEG, NSLAB, TW)
        dir3 = dirT[:, sl].reshape(DEG, NSLAB, 1)
        mask3 = maskT[:, sl].reshape(DEG, NSLAB, 1)
        outs.append(_post(
            xp[:, sl], g3, dir3, mask3, Wq.T, bq[None, :], Wo.T,
            bo[None, :], ln1_g[None, :], ln1_b[None, :], ln2_g[None, :],
            ln2_b[None, :], Wf1.T, bf1[None, :], Wf2.T, bf2[None, :],
            weC, beC, P))

    return jnp.concatenate(outs, axis=1)[:, :N]


# speedup vs baseline: 1.0136x; 1.0115x over previous
import functools
import math

import jax
import jax.numpy as jnp
from jax import lax
from jax.experimental import pallas as pl
from jax.experimental.pallas import tpu as pltpu
from jax.experimental.pallas import tpu_sc as plsc

B, N, DEG, D, H = 2, 10000, 16, 128, 4
HD = D // H
NW = 32
CH = 64
TW = 2 * D

NSLAB = 5120
NPADH = DEG * NSLAB
NWH = 16
ROWS_PER_W = NPADH // NWH
NCH = ROWS_PER_W // CH

NB = 128
GB = NB * DEG
NBLK = NSLAB // NB

NBP = 2000
NPRE = N // NBP


def _pack_bf16(y):
    u = lax.bitcast_convert_type(y, jnp.uint32)
    r = (u + jnp.uint32(0x8000)) >> jnp.uint32(16)
    packed = r[:, :D // 2] | (r[:, D // 2:] << jnp.uint32(16))
    return lax.bitcast_convert_type(packed, jnp.int32)


def _unpack_bf16(gi):
    gu = lax.bitcast_convert_type(gi, jnp.uint32)
    lo = lax.bitcast_convert_type(gu << jnp.uint32(16), jnp.float32)
    hi = lax.bitcast_convert_type(gu & jnp.uint32(0xFFFF0000), jnp.float32)
    return jnp.concatenate([lo, hi], axis=1)


def _pre_body(x_ref, wkT_ref, bk_ref, wvT_ref, bv_ref, out_ref):
    wkT = wkT_ref[...]
    wvT = wvT_ref[...]
    hw = D // 2
    for b in range(B):
        xb = x_ref[b]
        out_ref[:, b * hw:(b + 1) * hw] = _pack_bf16(
            jnp.dot(xb, wkT, preferred_element_type=jnp.float32) + bk_ref[...])
        out_ref[:, D + b * hw:D + (b + 1) * hw] = _pack_bf16(
            jnp.dot(xb, wvT, preferred_element_type=jnp.float32) + bv_ref[...])


def _build_table(x, WkT, bk, WvT, bv):
    return pl.pallas_call(
        _pre_body,
        grid=(NPRE,),
        in_specs=[
            pl.BlockSpec((B, NBP, D), lambda i: (0, i, 0)),
            pl.BlockSpec((D, D), lambda i: (0, 0)),
            pl.BlockSpec((1, D), lambda i: (0, 0)),
            pl.BlockSpec((D, D), lambda i: (0, 0)),
            pl.BlockSpec((1, D), lambda i: (0, 0)),
        ],
        out_specs=pl.BlockSpec((NBP, TW), lambda i: (i, 0)),
        out_shape=jax.ShapeDtypeStruct((N, TW), jnp.int32),
    )(x, WkT, bk, WvT, bv)


NBUF = 6


def _gather_body(table_hbm, idx_hbm, out_hbm, idx_v, rows_v, *sems):
    gsems = sems[:NBUF]
    wsems = sems[NBUF:]
    wid = lax.axis_index("s")
    base = wid * ROWS_PER_W
    pltpu.sync_copy(idx_hbm.at[pl.ds(base, ROWS_PER_W)], idx_v)
    bufs = [rows_v.at[i] for i in range(NBUF)]

    def start_g(j, b):
        pltpu.async_copy(
            table_hbm.at[idx_v.at[pl.ds(j * CH, CH)]], bufs[b], gsems[b])

    def wait_g(b):
        pltpu.make_async_copy(
            table_hbm.at[idx_v.at[pl.ds(0, CH)]], bufs[b], gsems[b]).wait()

    def start_w(j, b):
        pltpu.async_copy(
            bufs[b], out_hbm.at[pl.ds(base + j * CH, CH)], wsems[b])

    def wait_w(b):
        pltpu.make_async_copy(
            bufs[b], out_hbm.at[pl.ds(base, CH)], wsems[b]).wait()

    for b in range(NBUF - 1):
        start_g(b, b)
    for j in range(NCH):
        b = j % NBUF
        wait_g(b)
        start_w(j, b)
        jn = j + NBUF - 1
        if jn < NCH:
            bn = jn % NBUF
            if jn >= NBUF:
                wait_w(bn)
            start_g(jn, bn)
    for j in range(max(0, NCH - NBUF), NCH):
        wait_w(j % NBUF)


def _gather_rows(table, idx_pad):
    mesh = plsc.VectorSubcoreMesh(core_axis_name="c", subcore_axis_name="s",
                                  num_cores=1)
    k = pl.kernel(
        _gather_body,
        out_type=jax.ShapeDtypeStruct((NPADH, TW), jnp.int32),
        mesh=mesh,
        scratch_types=(
            [pltpu.VMEM((ROWS_PER_W,), jnp.int32),
             pltpu.VMEM((NBUF, CH, TW), jnp.int32)]
            + [pltpu.SemaphoreType.DMA] * (2 * NBUF)
        ),
    )
    return k(table, idx_pad)


def _post_body(x_ref, g_ref, dir_ref, mask_ref,
               wqT_ref, bq_ref, woT_ref, bo_ref,
               ln1g_ref, ln1b_ref, ln2g_ref, ln2b_ref,
               wf1T_ref, bf1_ref, wf2T_ref, bf2_ref,
               weC_ref, beC_ref, p_ref, out_ref):
    inv = 1.0 / math.sqrt(HD)
    hw = D // 2
    P = p_ref[...]
    P_lo = P[:hw, :]
    P_hi = P[hw:, :]
    woT = woT_ref[...]

    dirS = dir_ref[...].reshape(GB, 1)
    maskS = mask_ref[...].reshape(GB, 1)
    em = jnp.where(maskS > 0.5,
                   dirS * weC_ref[...] + beC_ref[...], -1e9)
    wqT = wqT_ref[...]
    gb = g_ref[...].reshape(GB, TW)

    def _halves(u):
        uu = lax.bitcast_convert_type(u, jnp.uint32)
        lo = lax.bitcast_convert_type(uu << jnp.uint32(16), jnp.float32)
        hi = lax.bitcast_convert_type(
            uu & jnp.uint32(0xFFFF0000), jnp.float32)
        return lo, hi

    def _slabsum(a):
        parts = [a[d * NB:(d + 1) * NB, :] for d in range(DEG)]
        while len(parts) > 1:
            parts = [parts[i] + parts[i + 1] for i in range(0, len(parts), 2)]
        return parts[0]

    for b in range(B):
        xb = x_ref[b]
        q = (jnp.dot(xb, wqT, preferred_element_type=jnp.float32)
             + bq_ref[...]) * inv
        q_lo = q[:, :hw]
        q_hi = q[:, hw:]
        kl, kh = _halves(gb[:, b * hw:(b + 1) * hw])
        prod_lo = (kl.reshape(DEG, NB, hw) * q_lo[None]).reshape(GB, hw)
        prod_hi = (kh.reshape(DEG, NB, hw) * q_hi[None]).reshape(GB, hw)
        lg = (jnp.dot(prod_lo, P_lo, preferred_element_type=jnp.float32)
              + jnp.dot(prod_hi, P_hi, preferred_element_type=jnp.float32)
              + em)
        e = jnp.exp(lg)
        vl, vh = _halves(gb[:, D + b * hw:D + (b + 1) * hw])
        wl = e[:, :hw] * vl
        wh = e[:, hw:] * vh
        den = _slabsum(e) + 1e-20
        agg_lo = _slabsum(wl) / den[:, :hw]
        agg_hi = _slabsum(wh) / den[:, hw:]

        h1 = (xb
              + jnp.dot(agg_lo, woT[:hw, :],
                        preferred_element_type=jnp.float32)
              + jnp.dot(agg_hi, woT[hw:, :],
                        preferred_element_type=jnp.float32) + bo_ref[...])
        m = jnp.mean(h1, axis=-1, keepdims=True)
        v = jnp.mean((h1 - m) ** 2, axis=-1, keepdims=True)
        h = (h1 - m) / jnp.sqrt(v + 1e-5) * ln1g_ref[...] + ln1b_ref[...]

        f = jnp.maximum(
            jnp.dot(h, wf1T_ref[...], preferred_element_type=jnp.float32)
            + bf1_ref[...], 0.0)
        f = jnp.dot(f, wf2T_ref[...],
                    preferred_element_type=jnp.float32) + bf2_ref[...]
        h2 = h + f
        m2 = jnp.mean(h2, axis=-1, keepdims=True)
        v2 = jnp.mean((h2 - m2) ** 2, axis=-1, keepdims=True)
        out_ref[b] = ((h2 - m2) / jnp.sqrt(v2 + 1e-5) * ln2g_ref[...]
                      + ln2b_ref[...])


def _post(x, g, dirE, maskE, WqT, bq, WoT, bo, ln1g, ln1b, ln2g, ln2b,
          Wf1T, bf1, Wf2T, bf2, weC, beC, P):
    full = lambda i: (0, 0)
    return pl.pallas_call(
        _post_body,
        grid=(NBLK,),
        in_specs=[
            pl.BlockSpec((B, NB, D), lambda i: (0, i, 0)),
            pl.BlockSpec((DEG, NB, TW), lambda i: (0, i, 0)),
            pl.BlockSpec((DEG, NB, 1), lambda i: (0, i, 0)),
            pl.BlockSpec((DEG, NB, 1), lambda i: (0, i, 0)),
            pl.BlockSpec((D, D), full),
            pl.BlockSpec((1, D), full),
            pl.BlockSpec((D, D), full),
            pl.BlockSpec((1, D), full),
            pl.BlockSpec((1, D), full),
            pl.BlockSpec((1, D), full),
            pl.BlockSpec((1, D), full),
            pl.BlockSpec((1, D), full),
            pl.BlockSpec((D, 2 * D), full),
            pl.BlockSpec((1, 2 * D), full),
            pl.BlockSpec((2 * D, D), full),
            pl.BlockSpec((1, D), full),
            pl.BlockSpec((1, D), full),
            pl.BlockSpec((1, D), full),
            pl.BlockSpec((D, D), lambda i: (0, 0)),
        ],
        out_specs=pl.BlockSpec((B, NB, D), lambda i: (0, i, 0)),
        out_shape=jax.ShapeDtypeStruct((B, NSLAB, D), jnp.float32),
    )(x, g, dirE, maskE, WqT, bq, WoT, bo, ln1g, ln1b, ln2g, ln2b,
      Wf1T, bf1, Wf2T, bf2, weC, beC, P)


def kernel(x, incoming_idx, incoming_dir, incoming_mask,
           Wq, bq, Wk, bk, Wv, bv, We, be, Wo, bo,
           ln1_g, ln1_b, ln2_g, ln2_b, Wf1, bf1, Wf2, bf2):
    table = _build_table(x, Wk.T, bk[None, :], Wv.T, bv[None, :])

    npad2 = 2 * NSLAB - N
    idxT = jnp.pad(incoming_idx.T, ((0, 0), (0, npad2)))
    dirT = jnp.pad(incoming_dir[:, :, 0].T, ((0, 0), (0, npad2)))
    maskT = jnp.pad(incoming_mask.T.astype(jnp.float32),
                    ((0, 0), (0, npad2)))
    xp = jnp.pad(x, ((0, 0), (0, npad2), (0, 0)))

    cols = jnp.arange(D, dtype=jnp.int32)
    P = (cols[:, None] // HD == cols[None, :] // HD).astype(jnp.float32)
    weC = jnp.repeat(We[:, 0], HD)[None, :]
    beC = jnp.repeat(be, HD)[None, :]

    outs = []
    for h0 in (0, NSLAB):
        sl = slice(h0, h0 + NSLAB)
        g = _gather_rows(table, idxT[:, sl].reshape(-1))
        g3 = g.reshape(DEG, NSLAB, TW)
        dir3 = dirT[:, sl].reshape(DEG, NSLAB, 1)
        mask3 = maskT[:, sl].reshape(DEG, NSLAB, 1)
        outs.append(_post(
            xp[:, sl], g3, dir3, mask3, Wq.T, bq[None, :], Wo.T,
            bo[None, :], ln1_g[None, :], ln1_b[None, :], ln2_g[None, :],
            ln2_b[None, :], Wf1.T, bf1[None, :], Wf2.T, bf2[None, :],
            weC, beC, P))

    return jnp.concatenate(outs, axis=1)[:, :N]


# speedup vs baseline: 1.0311x; 1.0172x over previous
import functools
import math

import jax
import jax.numpy as jnp
from jax import lax
from jax.experimental import pallas as pl
from jax.experimental.pallas import tpu as pltpu
from jax.experimental.pallas import tpu_sc as plsc

B, N, DEG, D, H = 2, 10000, 16, 128, 4
HD = D // H
NDEG = N * DEG
NW = 32
ROWS_PER_W = 5120
NPAD = NW * ROWS_PER_W
CH = 128
NCH = ROWS_PER_W // CH
TW = 2 * D

NB = 200
GB = NB * DEG
NBLK = N // NB
NSLAB = NPAD // DEG

NBP = 2000
NPRE = N // NBP


def _pack_bf16(y):
    u = lax.bitcast_convert_type(y, jnp.uint32)
    r = (u + jnp.uint32(0x8000)) >> jnp.uint32(16)
    packed = r[:, :D // 2] | (r[:, D // 2:] << jnp.uint32(16))
    return lax.bitcast_convert_type(packed, jnp.int32)


def _unpack_bf16(gi):
    gu = lax.bitcast_convert_type(gi, jnp.uint32)
    lo = lax.bitcast_convert_type(gu << jnp.uint32(16), jnp.float32)
    hi = lax.bitcast_convert_type(gu & jnp.uint32(0xFFFF0000), jnp.float32)
    return jnp.concatenate([lo, hi], axis=1)


def _pre_body(x_ref, wkT_ref, bk_ref, wvT_ref, bv_ref, out_ref):
    wkT = wkT_ref[...]
    wvT = wvT_ref[...]
    hw = D // 2
    for b in range(B):
        xb = x_ref[b]
        out_ref[:, b * hw:(b + 1) * hw] = _pack_bf16(
            jnp.dot(xb, wkT, preferred_element_type=jnp.float32) + bk_ref[...])
        out_ref[:, D + b * hw:D + (b + 1) * hw] = _pack_bf16(
            jnp.dot(xb, wvT, preferred_element_type=jnp.float32) + bv_ref[...])


def _build_table(x, WkT, bk, WvT, bv):
    return pl.pallas_call(
        _pre_body,
        grid=(NPRE,),
        in_specs=[
            pl.BlockSpec((B, NBP, D), lambda i: (0, i, 0)),
            pl.BlockSpec((D, D), lambda i: (0, 0)),
            pl.BlockSpec((1, D), lambda i: (0, 0)),
            pl.BlockSpec((D, D), lambda i: (0, 0)),
            pl.BlockSpec((1, D), lambda i: (0, 0)),
        ],
        out_specs=pl.BlockSpec((NBP, TW), lambda i: (i, 0)),
        out_shape=jax.ShapeDtypeStruct((N, TW), jnp.int32),
    )(x, WkT, bk, WvT, bv)


def _gather_body(table_hbm, idx_hbm, out_hbm, idx_v, rows_v, sem0, sem1):
    c = lax.axis_index("c")
    s = lax.axis_index("s")
    wid = s * 2 + c
    base = wid * ROWS_PER_W
    pltpu.sync_copy(idx_hbm.at[pl.ds(base, ROWS_PER_W)], idx_v)
    sems = (sem0, sem1)
    bufs = (rows_v.at[0], rows_v.at[1])

    def start_g(j, b):
        pltpu.async_copy(
            table_hbm.at[idx_v.at[pl.ds(j * CH, CH)]], bufs[b], sems[b])

    def finish(j, b):
        pltpu.make_async_copy(
            table_hbm.at[idx_v.at[pl.ds(0, CH)]], bufs[b], sems[b]).wait()
        pltpu.sync_copy(bufs[b], out_hbm.at[pl.ds(base + j * CH, CH)])

    start_g(0, 0)

    def body(p, carry):
        j0 = p * 2
        start_g(j0 + 1, 1)
        finish(j0, 0)

        @pl.when(p < NCH // 2 - 1)
        def _():
            start_g(j0 + 2, 0)

        finish(j0 + 1, 1)
        return carry

    lax.fori_loop(0, NCH // 2, body, 0)


def _gather_rows(table, idx_pad):
    mesh = plsc.VectorSubcoreMesh(core_axis_name="c", subcore_axis_name="s")
    k = pl.kernel(
        _gather_body,
        out_type=jax.ShapeDtypeStruct((NPAD, TW), jnp.int32),
        mesh=mesh,
        scratch_types=[
            pltpu.VMEM((ROWS_PER_W,), jnp.int32),
            pltpu.VMEM((2, CH, TW), jnp.int32),
            pltpu.SemaphoreType.DMA,
            pltpu.SemaphoreType.DMA,
        ],
    )
    return k(table, idx_pad)


def _post_body(x_ref, g_ref, dir_ref, mask_ref,
               wqT_ref, bq_ref, woT_ref, bo_ref,
               ln1g_ref, ln1b_ref, ln2g_ref, ln2b_ref,
               wf1T_ref, bf1_ref, wf2T_ref, bf2_ref,
               weC_ref, beC_ref, p_ref, out_ref):
    inv = 1.0 / math.sqrt(HD)
    hw = D // 2
    P = p_ref[...]
    P_lo = P[:hw, :]
    P_hi = P[hw:, :]
    woT = woT_ref[...]

    dirS = dir_ref[...].reshape(GB, 1)
    maskS = mask_ref[...].reshape(GB, 1)
    em = jnp.where(maskS > 0.5,
                   dirS * weC_ref[...] + beC_ref[...], -1e9)
    wqT = wqT_ref[...]
    gb = g_ref[...].reshape(GB, TW)

    def _halves(u):
        uu = lax.bitcast_convert_type(u, jnp.uint32)
        lo = lax.bitcast_convert_type(uu << jnp.uint32(16), jnp.float32)
        hi = lax.bitcast_convert_type(
            uu & jnp.uint32(0xFFFF0000), jnp.float32)
        return lo, hi

    def _slabsum(a):
        parts = [a[d * NB:(d + 1) * NB, :] for d in range(DEG)]
        while len(parts) > 1:
            parts = [parts[i] + parts[i + 1] for i in range(0, len(parts), 2)]
        return parts[0]

    for b in range(B):
        xb = x_ref[b]
        q = (jnp.dot(xb, wqT, preferred_element_type=jnp.float32)
             + bq_ref[...]) * inv
        q_lo = q[:, :hw]
        q_hi = q[:, hw:]
        kl, kh = _halves(gb[:, b * hw:(b + 1) * hw])
        prod_lo = (kl.reshape(DEG, NB, hw) * q_lo[None]).reshape(GB, hw)
        prod_hi = (kh.reshape(DEG, NB, hw) * q_hi[None]).reshape(GB, hw)
        lg = (jnp.dot(prod_lo, P_lo, preferred_element_type=jnp.float32)
              + jnp.dot(prod_hi, P_hi, preferred_element_type=jnp.float32)
              + em)
        e = jnp.exp(lg)
        vl, vh = _halves(gb[:, D + b * hw:D + (b + 1) * hw])
        wl = e[:, :hw] * vl
        wh = e[:, hw:] * vh
        den = _slabsum(e) + 1e-20
        agg_lo = _slabsum(wl) / den[:, :hw]
        agg_hi = _slabsum(wh) / den[:, hw:]

        h1 = (xb
              + jnp.dot(agg_lo, woT[:hw, :],
                        preferred_element_type=jnp.float32)
              + jnp.dot(agg_hi, woT[hw:, :],
                        preferred_element_type=jnp.float32) + bo_ref[...])
        m = jnp.mean(h1, axis=-1, keepdims=True)
        v = jnp.mean((h1 - m) ** 2, axis=-1, keepdims=True)
        h = (h1 - m) / jnp.sqrt(v + 1e-5) * ln1g_ref[...] + ln1b_ref[...]

        f = jnp.maximum(
            jnp.dot(h, wf1T_ref[...], preferred_element_type=jnp.float32)
            + bf1_ref[...], 0.0)
        f = jnp.dot(f, wf2T_ref[...],
                    preferred_element_type=jnp.float32) + bf2_ref[...]
        h2 = h + f
        m2 = jnp.mean(h2, axis=-1, keepdims=True)
        v2 = jnp.mean((h2 - m2) ** 2, axis=-1, keepdims=True)
        out_ref[b] = ((h2 - m2) / jnp.sqrt(v2 + 1e-5) * ln2g_ref[...]
                      + ln2b_ref[...])


def _post(x, g, dirE, maskE, WqT, bq, WoT, bo, ln1g, ln1b, ln2g, ln2b,
          Wf1T, bf1, Wf2T, bf2, weC, beC, P):
    full = lambda i: (0, 0)
    return pl.pallas_call(
        _post_body,
        grid=(NBLK,),
        in_specs=[
            pl.BlockSpec((B, NB, D), lambda i: (0, i, 0)),
            pl.BlockSpec((DEG, NB, TW), lambda i: (0, i, 0)),
            pl.BlockSpec((DEG, NB, 1), lambda i: (0, i, 0)),
            pl.BlockSpec((DEG, NB, 1), lambda i: (0, i, 0)),
            pl.BlockSpec((D, D), full),
            pl.BlockSpec((1, D), full),
            pl.BlockSpec((D, D), full),
            pl.BlockSpec((1, D), full),
            pl.BlockSpec((1, D), full),
            pl.BlockSpec((1, D), full),
            pl.BlockSpec((1, D), full),
            pl.BlockSpec((1, D), full),
            pl.BlockSpec((D, 2 * D), full),
            pl.BlockSpec((1, 2 * D), full),
            pl.BlockSpec((2 * D, D), full),
            pl.BlockSpec((1, D), full),
            pl.BlockSpec((1, D), full),
            pl.BlockSpec((1, D), full),
            pl.BlockSpec((D, D), lambda i: (0, 0)),
        ],
        out_specs=pl.BlockSpec((B, NB, D), lambda i: (0, i, 0)),
        out_shape=jax.ShapeDtypeStruct((B, N, D), jnp.float32),
    )(x, g, dirE, maskE, WqT, bq, WoT, bo, ln1g, ln1b, ln2g, ln2b,
      Wf1T, bf1, Wf2T, bf2, weC, beC, P)


def kernel(x, incoming_idx, incoming_dir, incoming_mask,
           Wq, bq, Wk, bk, Wv, bv, We, be, Wo, bo,
           ln1_g, ln1_b, ln2_g, ln2_b, Wf1, bf1, Wf2, bf2):
    table = _build_table(x, Wk.T, bk[None, :], Wv.T, bv[None, :])

    idx_pad = jnp.pad(incoming_idx.T, ((0, 0), (0, NSLAB - N))).reshape(-1)
    g = _gather_rows(table, idx_pad)
    g3 = g.reshape(DEG, NSLAB, TW)

    dir3 = jnp.pad(incoming_dir[:, :, 0].T,
                   ((0, 0), (0, NSLAB - N))).reshape(DEG, NSLAB, 1)
    mask3 = jnp.pad(incoming_mask.T.astype(jnp.float32),
                    ((0, 0), (0, NSLAB - N))).reshape(DEG, NSLAB, 1)

    cols = jnp.arange(D, dtype=jnp.int32)
    P = (cols[:, None] // HD == cols[None, :] // HD).astype(jnp.float32)
    weC = jnp.repeat(We[:, 0], HD)[None, :]
    beC = jnp.repeat(be, HD)[None, :]

    return _post(x, g3, dir3, mask3, Wq.T, bq[None, :], Wo.T, bo[None, :],
                 ln1_g[None, :], ln1_b[None, :], ln2_g[None, :],
                 ln2_b[None, :], Wf1.T, bf1[None, :], Wf2.T, bf2[None, :],
                 weC, beC, P)
